# Initial kernel scaffold; baseline (speedup 1.0000x reference)
import jax
import jax.numpy as jnp
from jax.experimental import pallas as pl


def kernel(hidden_states, router_w, router_bias, Wg, Wu, Wd, Sg, Su, Sd):
    raise NotImplementedError("write your pallas kernel here")


# speedup vs baseline: 1.4931x; 1.4931x over previous
import jax
import jax.numpy as jnp
from jax import lax
from jax.experimental import pallas as pl
from jax.experimental.pallas import tpu as pltpu

_D = 1024
_FF = 512
_E = 8
_K = 2


def _router_body(x_ref, rw_ref, rb_ref, cw_ref):
    x = x_ref[...]
    rw = rw_ref[...]
    t = x.shape[0]
    logits = lax.dot_general(x, rw, (((1,), (1,)), ((), ())),
                             preferred_element_type=jnp.float32)
    scores = jax.nn.sigmoid(logits)
    sc = scores + rb_ref[...]
    idx8 = lax.broadcasted_iota(jnp.int32, (t, _E), 1)
    m1 = jnp.max(sc, axis=1, keepdims=True)
    i1 = jnp.min(jnp.where(sc >= m1, idx8, _E), axis=1, keepdims=True)
    w1 = jnp.sum(jnp.where(idx8 == i1, scores, 0.0), axis=1, keepdims=True)
    sc2 = jnp.where(idx8 == i1, -jnp.inf, sc)
    m2 = jnp.max(sc2, axis=1, keepdims=True)
    i2 = jnp.min(jnp.where(sc2 >= m2, idx8, _E), axis=1, keepdims=True)
    w2 = jnp.sum(jnp.where(idx8 == i2, scores, 0.0), axis=1, keepdims=True)
    denom = w1 + w2 + 1e-20
    cw = (jnp.where(idx8 == i1, w1 / denom, 0.0)
          + jnp.where(idx8 == i2, w2 / denom, 0.0))
    cw_ref[...] = jnp.concatenate([cw, jnp.ones((t, 1), jnp.float32)], axis=1)


def _moe_dense_body(cwt_ref, x_ref, wg_ref, wu_ref, wd_ref, out_ref):
    e = pl.program_id(0)
    x = x_ref[...]
    g = lax.dot_general(x, wg_ref[0], (((1,), (1,)), ((), ())),
                        preferred_element_type=jnp.float32)
    u = lax.dot_general(x, wu_ref[0], (((1,), (1,)), ((), ())),
                        preferred_element_type=jnp.float32)
    h = (g * jax.nn.sigmoid(g)) * u
    y = lax.dot_general(h, wd_ref[0], (((1,), (1,)), ((), ())),
                        preferred_element_type=jnp.float32)
    y = y * cwt_ref[0, 0, :][:, None]

    @pl.when(e == 0)
    def _():
        out_ref[...] = y

    @pl.when(e > 0)
    def _():
        out_ref[...] += y


def kernel(hidden_states, router_w, router_bias, Wg, Wu, Wd, Sg, Su, Sd):
    orig_shape = hidden_states.shape
    x = hidden_states.reshape(-1, _D)
    t = x.shape[0]

    cw = pl.pallas_call(
        _router_body,
        out_shape=jax.ShapeDtypeStruct((t, _E + 1), jnp.float32),
    )(x, router_w, router_bias.reshape(1, _E))

    cwt = cw.T.reshape(_E + 1, 1, t)
    wg_all = jnp.concatenate([Wg, Sg[None]], axis=0)
    wu_all = jnp.concatenate([Wu, Su[None]], axis=0)
    wd_all = jnp.concatenate([Wd, Sd[None]], axis=0)

    out = pl.pallas_call(
        _moe_dense_body,
        grid=(_E + 1,),
        in_specs=[
            pl.BlockSpec((1, 1, t), lambda e: (e, 0, 0)),
            pl.BlockSpec((t, _D), lambda e: (0, 0)),
            pl.BlockSpec((1, _FF, _D), lambda e: (e, 0, 0)),
            pl.BlockSpec((1, _FF, _D), lambda e: (e, 0, 0)),
            pl.BlockSpec((1, _D, _FF), lambda e: (e, 0, 0)),
        ],
        out_specs=pl.BlockSpec((t, _D), lambda e: (0, 0)),
        out_shape=jax.ShapeDtypeStruct((t, _D), jnp.float32),
        compiler_params=pltpu.CompilerParams(
            dimension_semantics=("arbitrary",),
        ),
    )(cwt, x, wg_all, wu_all, wd_all)

    return out.reshape(orig_shape)
